# baseline (speedup 1.0000x reference)
import functools

import jax
import jax.numpy as jnp
from jax import lax
from jax.experimental import pallas as pl
from jax.experimental.pallas import tpu as pltpu
from jax.experimental.pallas import tpu_sc as plsc

_BLK = 128
_L = 16


@functools.lru_cache(maxsize=None)
def _build(T, B, D):
    info = plsc.get_sparse_core_info()
    nc, ns = info.num_cores, info.num_subcores
    nw = nc * ns
    assert B == nw * _BLK and D % _L == 0
    n_pair = T // 2
    assert n_pair * 2 == T
    mesh = plsc.VectorSubcoreMesh(core_axis_name="c", subcore_axis_name="s")

    @functools.partial(
        pl.kernel,
        mesh=mesh,
        out_type=jax.ShapeDtypeStruct((T, D // 8, B // 128, 8, 128), jnp.float32),
        scratch_types=[
            pltpu.VMEM((2, _BLK), jnp.int32),
            pltpu.VMEM((2, _BLK, D), jnp.float32),
            pltpu.VMEM((2, D // 8, 1, 8, _BLK), jnp.float32),
            pltpu.SemaphoreType.DMA((2,)),
            pltpu.SemaphoreType.DMA((2,)),
            pltpu.SemaphoreType.DMA((2,)),
        ],
        compiler_params=pltpu.CompilerParams(
            use_tc_tiling_on_sc=False, needs_layout_passes=False),
    )
    def gather_kernel(xt_hbm, table_hbm, out_hbm, idx_v, rows_v, tr_v,
                      isem, gsem, osem):
        wid = lax.axis_index("s") * nc + lax.axis_index("c")
        boff = pl.multiple_of(wid * _BLK, _BLK)
        lane = jnp.arange(_L, dtype=jnp.int32)

        def load_idx(t, buf):
            pltpu.async_copy(
                xt_hbm.at[t, pl.ds(boff, _BLK)], idx_v.at[buf], isem.at[buf])

        def fire_gather(buf):
            pltpu.make_async_copy(
                xt_hbm.at[0, pl.ds(0, _BLK)], idx_v.at[buf],
                isem.at[buf]).wait()
            for k in range(_BLK // _L):
                v = idx_v[buf, pl.ds(k * _L, _L)]
                idx_v[buf, pl.ds(k * _L, _L)] = v + v
            pltpu.async_copy(
                table_hbm.at[idx_v.at[buf]], rows_v.at[buf], gsem.at[buf])

        load_idx(0, 0)
        load_idx(1, 1)
        fire_gather(0)

        def pair(g, carry):
            for b in range(2):
                t = g * 2 + b
                o = 1 - b

                pltpu.make_async_copy(
                    table_hbm.at[pl.ds(0, _BLK)], rows_v.at[b],
                    gsem.at[b]).wait()

                @pl.when(t + 2 < T)
                def _():
                    load_idx(t + 2, b)

                @pl.when(t + 1 < T)
                def _():
                    fire_gather(o)

                @pl.when(g > 0)
                def _():
                    pltpu.make_async_copy(
                        tr_v.at[b],
                        out_hbm.at[0, pl.ds(0, D // 8), pl.ds(0, 1),
                                   pl.ds(0, 8), pl.ds(0, _BLK)],
                        osem.at[b]).wait()

                def trans_d(d, carry2):
                    ds = jnp.full((_L,), d, dtype=jnp.int32)
                    for j in range(_BLK // _L):
                        col = plsc.load_gather(
                            rows_v.at[b], [lane + (j * _L), ds])
                        tr_v[b, d // 8, 0, d % 8, pl.ds(j * _L, _L)] = col
                    return carry2

                lax.fori_loop(0, D, trans_d, 0)

                pltpu.async_copy(
                    tr_v.at[b],
                    out_hbm.at[t, pl.ds(0, D // 8), pl.ds(wid, 1),
                               pl.ds(0, 8), pl.ds(0, _BLK)],
                    osem.at[b])
            return carry

        lax.fori_loop(0, n_pair, pair, 0)
        for b in range(2):
            pltpu.make_async_copy(
                tr_v.at[b],
                out_hbm.at[0, pl.ds(0, D // 8), pl.ds(0, 1),
                           pl.ds(0, 8), pl.ds(0, _BLK)],
                osem.at[b]).wait()

    return gather_kernel


def kernel(x, table):
    Bt, T = x.shape
    V, D = table.shape
    DP = 128
    xt = jnp.transpose(x).astype(jnp.int32)
    table_p = jnp.pad(table, ((0, 0), (0, DP - D)))
    t64 = table_p.reshape(2 * V, D)
    out5 = _build(T, Bt, D)(xt, t64)
    return jnp.transpose(out5, (2, 4, 0, 1, 3)).reshape(Bt, T, D)


# speedup vs baseline: 2.0366x; 2.0366x over previous
import functools

import jax
import jax.numpy as jnp
from jax import lax
from jax.experimental import pallas as pl
from jax.experimental.pallas import tpu as pltpu
from jax.experimental.pallas import tpu_sc as plsc

_CHUNK = 128
_SUPER = 640
_NBUF = 2
_LANES = 16


@functools.lru_cache(maxsize=None)
def _build(B, D, DP):
    info = plsc.get_sparse_core_info()
    nc, ns = info.num_cores, info.num_subcores
    nw = nc * ns
    per_w = B // nw
    n_super = per_w // _SUPER
    n_pair = n_super // _NBUF
    G = _SUPER // _CHUNK
    assert per_w * nw == B and n_super * _SUPER == per_w and n_pair * _NBUF == n_super
    mesh = plsc.VectorSubcoreMesh(core_axis_name="c", subcore_axis_name="s")

    @functools.partial(
        pl.kernel,
        mesh=mesh,
        out_type=jax.ShapeDtypeStruct((B, DP), jnp.float32),
        scratch_types=[
            pltpu.VMEM((_NBUF, _SUPER), jnp.int32),
            pltpu.VMEM((_NBUF, _SUPER, D), jnp.float32),
            pltpu.SemaphoreType.DMA((_NBUF,)),
            pltpu.SemaphoreType.DMA((_NBUF,)),
            pltpu.SemaphoreType.DMA((_NBUF,)),
        ],
        compiler_params=pltpu.CompilerParams(use_tc_tiling_on_sc=False),
    )
    def gather_kernel(idx_hbm, table_hbm, out_hbm, idx_v, rows_v, isem, gsem, osem):
        wid = lax.axis_index("s") * nc + lax.axis_index("c")
        base = wid * per_w

        def idx_off(i):
            return pl.multiple_of(base + i * _SUPER, _SUPER)

        for b in range(_NBUF):
            pltpu.async_copy(
                idx_hbm.at[pl.ds(idx_off(b), _SUPER)], idx_v.at[b], isem.at[b])

        def pair(g, carry):
            for b in range(_NBUF):
                i = g * _NBUF + b

                @pl.when(g > 0)
                def _():
                    pltpu.make_async_copy(
                        rows_v.at[b],
                        out_hbm.at[pl.ds(idx_off(0), _SUPER), pl.ds(0, D)],
                        osem.at[b]).wait()

                pltpu.make_async_copy(
                    idx_hbm.at[pl.ds(idx_off(0), _SUPER)], idx_v.at[b],
                    isem.at[b]).wait()

                for k in range(_SUPER // _LANES):
                    v = idx_v[b, pl.ds(k * _LANES, _LANES)]
                    idx_v[b, pl.ds(k * _LANES, _LANES)] = v + v

                for j in range(G):
                    pltpu.async_copy(
                        table_hbm.at[idx_v.at[b, pl.ds(j * _CHUNK, _CHUNK)]],
                        rows_v.at[b, pl.ds(j * _CHUNK, _CHUNK)],
                        gsem.at[b])

                pltpu.make_async_copy(
                    table_hbm.at[pl.ds(0, _SUPER)], rows_v.at[b],
                    gsem.at[b]).wait()

                @pl.when(g < n_pair - 1)
                def _():
                    pltpu.async_copy(
                        idx_hbm.at[pl.ds(idx_off(i + _NBUF), _SUPER)],
                        idx_v.at[b], isem.at[b])

                pltpu.async_copy(
                    rows_v.at[b],
                    out_hbm.at[pl.ds(idx_off(i), _SUPER), pl.ds(0, D)],
                    osem.at[b])
            return carry

        lax.fori_loop(0, n_pair, pair, 0)
        for b in range(_NBUF):
            pltpu.make_async_copy(
                rows_v.at[b],
                out_hbm.at[pl.ds(idx_off(0), _SUPER), pl.ds(0, D)],
                osem.at[b]).wait()

    return gather_kernel


def kernel(x, table):
    B = x.shape[0] * x.shape[1]
    V, D = table.shape
    DP = 128
    idx = x.reshape(B).astype(jnp.int32)
    table_p = jnp.pad(table, ((0, 0), (0, DP - D)))
    t64 = table_p.reshape(2 * V, D)
    out = _build(B, D, DP)(idx, t64)
    return out[:, :D].reshape(x.shape + (D,))
